# baseline (speedup 1.0000x reference)
import functools

import jax
import jax.numpy as jnp
from jax import lax
from jax.experimental import pallas as pl
from jax.experimental.pallas import tpu as pltpu
from jax.experimental.pallas import tpu_sc as plsc

NQ = 1024
SEQ = 20
D = 128
ND = 100000
K = 10

CHUNK = 160
NCHUNK = ND // CHUNK
QBLK = 256
NQBLK = NQ // QBLK
DBLK = 2560
CPB = DBLK // CHUNK
NDBLK = (ND + DBLK - 1) // DBLK
NCHUNK_PAD = NDBLK * CPB

_BIG = 1 << 30
_NEG = -jnp.inf
IDS_PAD = 128


def _project_body(q_ref, w_ref, b_ref, qn_ref):
    pooled = jnp.mean(q_ref[...], axis=1)
    proj = lax.dot_general(pooled, w_ref[...], (((1,), (1,)), ((), ())))
    proj = proj + b_ref[...]
    nrm = jnp.sqrt(jnp.sum(proj * proj, axis=1, keepdims=True))
    nrm = jnp.maximum(nrm, 1e-12)
    qn_ref[...] = proj / nrm


def _project(q, w, b2d):
    return pl.pallas_call(
        _project_body,
        grid=(NQBLK,),
        in_specs=[
            pl.BlockSpec((QBLK, SEQ, D), lambda i: (i, 0, 0)),
            pl.BlockSpec((D, D), lambda i: (0, 0)),
            pl.BlockSpec((1, D), lambda i: (0, 0)),
        ],
        out_specs=pl.BlockSpec((QBLK, D), lambda i: (i, 0)),
        out_shape=jax.ShapeDtypeStruct((NQ, D), jnp.float32),
    )(q, w, b2d)


def _sim_body(qn_ref, doc_ref, sim_ref, mx_ref):
    di = pl.program_id(0)
    s = lax.dot_general(qn_ref[...], doc_ref[...], (((1,), (1,)), ((), ())))
    sim_ref[...] = s
    lane = lax.broadcasted_iota(jnp.int32, (QBLK, DBLK), 1)
    valid = (di * DBLK + lane) < ND
    sm = jnp.where(valid, s, _NEG)
    maxes = [jnp.max(sm[:, c * CHUNK:(c + 1) * CHUNK], axis=1)
             for c in range(CPB)]
    mx_ref[...] = jnp.stack(maxes, axis=1)[None]


def _sim_and_maxima(qn, doc):
    return pl.pallas_call(
        _sim_body,
        grid=(NDBLK, NQBLK),
        in_specs=[
            pl.BlockSpec((QBLK, D), lambda di, qi: (qi, 0)),
            pl.BlockSpec((DBLK, D), lambda di, qi: (di, 0)),
        ],
        out_specs=[
            pl.BlockSpec((QBLK, DBLK), lambda di, qi: (qi, di)),
            pl.BlockSpec((1, QBLK, CPB), lambda di, qi: (di, qi, 0)),
        ],
        out_shape=[
            jax.ShapeDtypeStruct((NQ, ND), jnp.float32),
            jax.ShapeDtypeStruct((NDBLK, NQ, CPB), jnp.float32),
        ],
    )(qn, doc)


def _select_body(mx_ref, ids_ref):
    v = mx_ref[...]
    cid = (lax.broadcasted_iota(jnp.int32, v.shape, 0) * CPB
           + lax.broadcasted_iota(jnp.int32, v.shape, 2))
    picks = []
    for _ in range(K):
        m = jnp.max(jnp.max(v, axis=0), axis=1)
        cand = jnp.where(v == m[None, :, None], cid, _BIG)
        sel = jnp.min(jnp.min(cand, axis=0), axis=1)
        picks.append(sel)
        v = jnp.where(cid == sel[None, :, None], _NEG, v)
    idmat = jnp.stack(picks, axis=1)
    outs = []
    for _ in range(K):
        mn = jnp.min(idmat, axis=1)
        outs.append(mn)
        idmat = jnp.where(idmat == mn[:, None], _BIG, idmat)
    sorted_ids = jnp.stack(outs, axis=1)
    pad = jnp.zeros((QBLK, IDS_PAD - K), jnp.int32)
    ids_ref[...] = jnp.concatenate([sorted_ids, pad], axis=1)


def _select_chunks(mx):
    return pl.pallas_call(
        _select_body,
        grid=(NQBLK,),
        in_specs=[pl.BlockSpec((NDBLK, QBLK, CPB), lambda qi: (0, qi, 0))],
        out_specs=pl.BlockSpec((QBLK, IDS_PAD), lambda qi: (qi, 0)),
        out_shape=jax.ShapeDtypeStruct((NQ, IDS_PAD), jnp.int32),
    )(mx)


_SC_NC = 2
_SC_NS = 16
_NW = _SC_NC * _SC_NS
_B = NQ * K
_BPW = _B // _NW
_GCHUNKS = [(o, min(128, _BPW - o)) for o in range(0, _BPW, 128)]


_RPW = NQ // _NW


_GW = 256


_KP = 16


def _gather_body(sim_hbm, ids_hbm, out_hbm, ids_v, buf8_v, cand_v, sem):
    c = lax.axis_index("c")
    s = lax.axis_index("s")
    wid = s * _SC_NC + c
    base = wid * _RPW
    pltpu.sync_copy(ids_hbm.at[pl.ds(base, _RPW)], ids_v)

    for g in range(_RPW // 8):
        def row(r8, carry):
            r_loc = g * 8 + r8
            row0 = pl.multiple_of(base + g * 8, 8)
            rvec = ids_v[r_loc, pl.ds(0, 16)]
            rems = []
            cps = []
            for j in range(K):
                q = rvec[j] * CHUNK
                off_al = pl.multiple_of((q // 128) * 128, 128)
                rems.append(q - off_al)
                cps.append(pltpu.make_async_copy(
                    sim_hbm.at[pl.ds(row0, 8), pl.ds(off_al, _GW)],
                    buf8_v.at[j],
                    sem,
                ))
            for cp in cps:
                cp.start()
            for cp in cps:
                cp.wait()
            for j in range(K):
                for v in range(CHUNK // 16):
                    cand_v[r8, j, pl.ds(v * 16, 16)] = (
                        buf8_v[j, r8, pl.ds(rems[j] + v * 16, 16)])
            return carry

        lax.fori_loop(0, 8, row, 0)
        pltpu.sync_copy(cand_v, out_hbm.at[pl.ds(base + g * 8, 8)])


@functools.cache
def _gather_candidates():
    return pl.kernel(
        _gather_body,
        out_type=jax.ShapeDtypeStruct((NQ, _KP, _GW), jnp.float32),
        mesh=plsc.VectorSubcoreMesh(
            core_axis_name="c", subcore_axis_name="s",
            num_cores=_SC_NC, num_subcores=_SC_NS,
        ),
        scratch_types=[
            pltpu.VMEM((_RPW, IDS_PAD), jnp.int32),
            pltpu.VMEM((K, 8, _GW), jnp.float32),
            pltpu.VMEM((8, _KP, _GW), jnp.float32),
            pltpu.SemaphoreType.DMA,
        ],
    )


NCAND = K * CHUNK


def _final_body(cand_ref, ids_ref, out_ref):
    v = cand_ref[...][:, :K, :CHUNK]
    ids = ids_ref[...][:, :K]
    shp = (QBLK, K, CHUNK)
    p_iota = (lax.broadcasted_iota(jnp.int32, shp, 1) * CHUNK
              + lax.broadcasted_iota(jnp.int32, shp, 2))
    j_iota = lax.broadcasted_iota(jnp.int32, (QBLK, K), 1)
    outs = []
    for _ in range(K):
        m = jnp.max(jnp.max(v, axis=1), axis=1)
        candp = jnp.where(v == m[:, None, None], p_iota, _BIG)
        p = jnp.min(jnp.min(candp, axis=1), axis=1)
        jj = p // CHUNK
        lane = p - jj * CHUNK
        cidsel = jnp.sum(jnp.where(j_iota == jj[:, None], ids, 0), axis=1)
        outs.append(cidsel * CHUNK + lane)
        v = jnp.where(p_iota == p[:, None, None], _NEG, v)
    out_ref[...] = jnp.stack(outs, axis=1)


def _final_topk(cand, ids):
    return pl.pallas_call(
        _final_body,
        grid=(NQBLK,),
        in_specs=[
            pl.BlockSpec((QBLK, _KP, _GW), lambda qi: (qi, 0, 0)),
            pl.BlockSpec((QBLK, IDS_PAD), lambda qi: (qi, 0)),
        ],
        out_specs=pl.BlockSpec((QBLK, K), lambda qi: (qi, 0)),
        out_shape=jax.ShapeDtypeStruct((NQ, K), jnp.int32),
    )(cand, ids)


def kernel(query_embeddings, W, b, doc_embeddings):
    qn = _project(query_embeddings, W, b.reshape(1, D))
    sim, mx = _sim_and_maxima(qn, doc_embeddings)
    ids = _select_chunks(mx)
    cand = _gather_candidates()(sim, ids)
    topk = _final_topk(cand, ids)
    return (topk, sim)


# speedup vs baseline: 1.2469x; 1.2469x over previous
import functools

import jax
import jax.numpy as jnp
from jax import lax
from jax.experimental import pallas as pl
from jax.experimental.pallas import tpu as pltpu
from jax.experimental.pallas import tpu_sc as plsc

NQ = 1024
SEQ = 20
D = 128
ND = 100000
K = 10

CHUNK = 160
NCHUNK = ND // CHUNK
QBLK = 256
NQBLK = NQ // QBLK
DBLK = 2560
CPB = DBLK // CHUNK
NDBLK = (ND + DBLK - 1) // DBLK
NCHUNK_PAD = NDBLK * CPB

_BIG = 1 << 30
_NEG = -jnp.inf
IDS_PAD = 128


def _project_body(q_ref, w_ref, b_ref, qn_ref):
    pooled = jnp.mean(q_ref[...], axis=1)
    proj = lax.dot_general(pooled, w_ref[...], (((1,), (1,)), ((), ())))
    proj = proj + b_ref[...]
    nrm = jnp.sqrt(jnp.sum(proj * proj, axis=1, keepdims=True))
    nrm = jnp.maximum(nrm, 1e-12)
    qn_ref[...] = proj / nrm


def _project(q, w, b2d):
    return pl.pallas_call(
        _project_body,
        grid=(NQBLK,),
        in_specs=[
            pl.BlockSpec((QBLK, SEQ, D), lambda i: (i, 0, 0)),
            pl.BlockSpec((D, D), lambda i: (0, 0)),
            pl.BlockSpec((1, D), lambda i: (0, 0)),
        ],
        out_specs=pl.BlockSpec((QBLK, D), lambda i: (i, 0)),
        out_shape=jax.ShapeDtypeStruct((NQ, D), jnp.float32),
    )(q, w, b2d)


def _sim_body(qn_ref, doc_ref, sim_ref, mx_ref):
    di = pl.program_id(0)
    s = lax.dot_general(qn_ref[...], doc_ref[...], (((1,), (1,)), ((), ())))
    sim_ref[...] = s
    lane = lax.broadcasted_iota(jnp.int32, (QBLK, DBLK), 1)
    valid = (di * DBLK + lane) < ND
    sm = jnp.where(valid, s, _NEG)
    maxes = [jnp.max(sm[:, c * CHUNK:(c + 1) * CHUNK], axis=1)
             for c in range(CPB)]
    mx_ref[...] = jnp.stack(maxes, axis=0)


def _sim_and_maxima(qn, doc):
    return pl.pallas_call(
        _sim_body,
        grid=(NDBLK, NQBLK),
        in_specs=[
            pl.BlockSpec((QBLK, D), lambda di, qi: (qi, 0)),
            pl.BlockSpec((DBLK, D), lambda di, qi: (di, 0)),
        ],
        out_specs=[
            pl.BlockSpec((QBLK, DBLK), lambda di, qi: (qi, di)),
            pl.BlockSpec((CPB, QBLK), lambda di, qi: (di, qi)),
        ],
        out_shape=[
            jax.ShapeDtypeStruct((NQ, ND), jnp.float32),
            jax.ShapeDtypeStruct((NCHUNK_PAD, NQ), jnp.float32),
        ],
    )(qn, doc)


def _select_body(mx_ref, ids_ref):
    v = mx_ref[...]
    cid = lax.broadcasted_iota(jnp.int32, v.shape, 0)
    picks = []
    for _ in range(K):
        m = jnp.max(v, axis=0)
        cand = jnp.where(v == m[None, :], cid, _BIG)
        sel = jnp.min(cand, axis=0)
        picks.append(sel)
        v = jnp.where(cid == sel[None, :], _NEG, v)
    idmat = jnp.stack(picks, axis=0)
    outs = []
    for _ in range(K):
        mn = jnp.min(idmat, axis=0)
        outs.append(mn)
        idmat = jnp.where(idmat == mn[None, :], _BIG, idmat)
    sorted_ids = jnp.stack(outs, axis=1)
    pad = jnp.zeros((QBLK, IDS_PAD - K), jnp.int32)
    ids_ref[...] = jnp.concatenate([sorted_ids, pad], axis=1)


def _select_chunks(mx):
    return pl.pallas_call(
        _select_body,
        grid=(NQBLK,),
        in_specs=[pl.BlockSpec((NCHUNK_PAD, QBLK), lambda qi: (0, qi))],
        out_specs=pl.BlockSpec((QBLK, IDS_PAD), lambda qi: (qi, 0)),
        out_shape=jax.ShapeDtypeStruct((NQ, IDS_PAD), jnp.int32),
    )(mx)


_SC_NC = 2
_SC_NS = 16
_NW = _SC_NC * _SC_NS
_B = NQ * K
_BPW = _B // _NW
_GCHUNKS = [(o, min(128, _BPW - o)) for o in range(0, _BPW, 128)]


_RPW = NQ // _NW


_GW = 256


NCAND = K * CHUNK
CAND_W = 1792


def _gather_body(sim_hbm, ids_hbm, out_hbm, ids_v, buf8_v, cand_v, sem):
    c = lax.axis_index("c")
    s = lax.axis_index("s")
    wid = s * _SC_NC + c
    base = wid * _RPW
    pltpu.sync_copy(ids_hbm.at[pl.ds(base, _RPW)], ids_v)

    for g in range(_RPW // 8):
        def row(r8, carry):
            r_loc = g * 8 + r8
            row0 = pl.multiple_of(base + g * 8, 8)
            rvec = ids_v[r_loc, pl.ds(0, 16)]
            rems = []
            cps = []
            for j in range(K):
                q = rvec[j] * CHUNK
                off_al = pl.multiple_of((q // 128) * 128, 128)
                rems.append(q - off_al)
                cps.append(pltpu.make_async_copy(
                    sim_hbm.at[pl.ds(row0, 8), pl.ds(off_al, _GW)],
                    buf8_v.at[j],
                    sem,
                ))
            for cp in cps:
                cp.start()
            for cp in cps:
                cp.wait()
            for j in range(K):
                for v in range(CHUNK // 16):
                    cand_v[r8, pl.ds(j * CHUNK + v * 16, 16)] = (
                        buf8_v[j, r8, pl.ds(rems[j] + v * 16, 16)])
            return carry

        lax.fori_loop(0, 8, row, 0)
        pltpu.sync_copy(cand_v, out_hbm.at[pl.ds(base + g * 8, 8)])


@functools.cache
def _gather_candidates():
    return pl.kernel(
        _gather_body,
        out_type=jax.ShapeDtypeStruct((NQ, CAND_W), jnp.float32),
        mesh=plsc.VectorSubcoreMesh(
            core_axis_name="c", subcore_axis_name="s",
            num_cores=_SC_NC, num_subcores=_SC_NS,
        ),
        scratch_types=[
            pltpu.VMEM((_RPW, IDS_PAD), jnp.int32),
            pltpu.VMEM((K, 8, _GW), jnp.float32),
            pltpu.VMEM((8, CAND_W), jnp.float32),
            pltpu.SemaphoreType.DMA,
        ],
    )


NCAND = K * CHUNK


def _final_body(cand_ref, ids_ref, out_ref):
    ids = ids_ref[...][:, :K]
    p_iota = lax.broadcasted_iota(jnp.int32, (QBLK, CAND_W), 1)
    v = jnp.where(p_iota < NCAND, cand_ref[...], _NEG)
    j_iota = lax.broadcasted_iota(jnp.int32, (QBLK, K), 1)
    outs = []
    for _ in range(K):
        m = jnp.max(v, axis=1)
        candp = jnp.where(v == m[:, None], p_iota, _BIG)
        p = jnp.min(candp, axis=1)
        jj = p // CHUNK
        lane = p - jj * CHUNK
        cidsel = jnp.sum(jnp.where(j_iota == jj[:, None], ids, 0), axis=1)
        outs.append(cidsel * CHUNK + lane)
        v = jnp.where(p_iota == p[:, None], _NEG, v)
    out_ref[...] = jnp.stack(outs, axis=1)


def _final_topk(cand, ids):
    return pl.pallas_call(
        _final_body,
        grid=(NQBLK,),
        in_specs=[
            pl.BlockSpec((QBLK, CAND_W), lambda qi: (qi, 0)),
            pl.BlockSpec((QBLK, IDS_PAD), lambda qi: (qi, 0)),
        ],
        out_specs=pl.BlockSpec((QBLK, K), lambda qi: (qi, 0)),
        out_shape=jax.ShapeDtypeStruct((NQ, K), jnp.int32),
    )(cand, ids)


def kernel(query_embeddings, W, b, doc_embeddings):
    qn = _project(query_embeddings, W, b.reshape(1, D))
    sim, mx = _sim_and_maxima(qn, doc_embeddings)
    ids = _select_chunks(mx)
    cand = _gather_candidates()(sim, ids)
    topk = _final_topk(cand, ids)
    return (topk, sim)


# speedup vs baseline: 1.4080x; 1.1292x over previous
import functools

import jax
import jax.numpy as jnp
from jax import lax
from jax.experimental import pallas as pl
from jax.experimental.pallas import tpu as pltpu
from jax.experimental.pallas import tpu_sc as plsc

NQ = 1024
SEQ = 20
D = 128
ND = 100000
K = 10

CHUNK = 160
NCHUNK = ND // CHUNK
QBLK = 512
NQBLK = NQ // QBLK
DBLK = 5120
CPB = DBLK // CHUNK
NDBLK = (ND + DBLK - 1) // DBLK
NCHUNK_PAD = NDBLK * CPB

_BIG = 1 << 30
_NEG = -jnp.inf
IDS_PAD = 128


def _project_body(q_ref, w_ref, b_ref, qn_ref):
    pooled = jnp.mean(q_ref[...], axis=1)
    proj = lax.dot_general(pooled, w_ref[...], (((1,), (1,)), ((), ())))
    proj = proj + b_ref[...]
    nrm = jnp.sqrt(jnp.sum(proj * proj, axis=1, keepdims=True))
    nrm = jnp.maximum(nrm, 1e-12)
    qn_ref[...] = proj / nrm


def _project(q, w, b2d):
    return pl.pallas_call(
        _project_body,
        grid=(NQBLK,),
        in_specs=[
            pl.BlockSpec((QBLK, SEQ, D), lambda i: (i, 0, 0)),
            pl.BlockSpec((D, D), lambda i: (0, 0)),
            pl.BlockSpec((1, D), lambda i: (0, 0)),
        ],
        out_specs=pl.BlockSpec((QBLK, D), lambda i: (i, 0)),
        out_shape=jax.ShapeDtypeStruct((NQ, D), jnp.float32),
    )(q, w, b2d)


def _sim_body(qn_ref, doc_ref, sim_ref, mx_ref):
    di = pl.program_id(0)
    s = lax.dot_general(qn_ref[...], doc_ref[...], (((1,), (1,)), ((), ())))
    sim_ref[...] = s
    lane = lax.broadcasted_iota(jnp.int32, (QBLK, DBLK), 1)
    valid = (di * DBLK + lane) < ND
    sm = jnp.where(valid, s, _NEG)
    maxes = [jnp.max(sm[:, c * CHUNK:(c + 1) * CHUNK], axis=1)
             for c in range(CPB)]
    mx_ref[...] = jnp.stack(maxes, axis=0)


def _sim_and_maxima(qn, doc):
    return pl.pallas_call(
        _sim_body,
        grid=(NDBLK, NQBLK),
        in_specs=[
            pl.BlockSpec((QBLK, D), lambda di, qi: (qi, 0)),
            pl.BlockSpec((DBLK, D), lambda di, qi: (di, 0)),
        ],
        out_specs=[
            pl.BlockSpec((QBLK, DBLK), lambda di, qi: (qi, di)),
            pl.BlockSpec((CPB, QBLK), lambda di, qi: (di, qi)),
        ],
        out_shape=[
            jax.ShapeDtypeStruct((NQ, ND), jnp.float32),
            jax.ShapeDtypeStruct((NCHUNK_PAD, NQ), jnp.float32),
        ],
    )(qn, doc)


def _select_body(mx_ref, ids_ref):
    v = mx_ref[...]
    cid = lax.broadcasted_iota(jnp.int32, v.shape, 0)
    picks = []
    for _ in range(K):
        m = jnp.max(v, axis=0)
        cand = jnp.where(v == m[None, :], cid, _BIG)
        sel = jnp.min(cand, axis=0)
        picks.append(sel)
        v = jnp.where(cid == sel[None, :], _NEG, v)
    idmat = jnp.stack(picks, axis=0)
    outs = []
    for _ in range(K):
        mn = jnp.min(idmat, axis=0)
        outs.append(mn)
        idmat = jnp.where(idmat == mn[None, :], _BIG, idmat)
    sorted_ids = jnp.stack(outs, axis=1)
    pad = jnp.zeros((QBLK, IDS_PAD - K), jnp.int32)
    ids_ref[...] = jnp.concatenate([sorted_ids, pad], axis=1)


def _select_chunks(mx):
    return pl.pallas_call(
        _select_body,
        grid=(NQBLK,),
        in_specs=[pl.BlockSpec((NCHUNK_PAD, QBLK), lambda qi: (0, qi))],
        out_specs=pl.BlockSpec((QBLK, IDS_PAD), lambda qi: (qi, 0)),
        out_shape=jax.ShapeDtypeStruct((NQ, IDS_PAD), jnp.int32),
    )(mx)


_SC_NC = 2
_SC_NS = 16
_NW = _SC_NC * _SC_NS
_B = NQ * K
_BPW = _B // _NW
_GCHUNKS = [(o, min(128, _BPW - o)) for o in range(0, _BPW, 128)]


_RPW = NQ // _NW


_GW = 256


NCAND = K * CHUNK
CAND_W = 1792


def _gather_body(sim_hbm, ids_hbm, out_hbm, ids_v, buf8_v, cand_v, sem):
    c = lax.axis_index("c")
    s = lax.axis_index("s")
    wid = s * _SC_NC + c
    base = wid * _RPW
    pltpu.sync_copy(ids_hbm.at[pl.ds(base, _RPW)], ids_v)

    for g in range(_RPW // 8):
        def row(r8, carry):
            r_loc = g * 8 + r8
            row0 = pl.multiple_of(base + g * 8, 8)
            rvec = ids_v[r_loc, pl.ds(0, 16)]
            rems = []
            cps = []
            for j in range(K):
                q = rvec[j] * CHUNK
                off_al = pl.multiple_of((q // 128) * 128, 128)
                rems.append(q - off_al)
                cps.append(pltpu.make_async_copy(
                    sim_hbm.at[pl.ds(row0, 8), pl.ds(off_al, _GW)],
                    buf8_v.at[j],
                    sem,
                ))
            for cp in cps:
                cp.start()
            for cp in cps:
                cp.wait()
            for j in range(K):
                for v in range(CHUNK // 16):
                    cand_v[r8, pl.ds(j * CHUNK + v * 16, 16)] = (
                        buf8_v[j, r8, pl.ds(rems[j] + v * 16, 16)])
            return carry

        lax.fori_loop(0, 8, row, 0)
        pltpu.sync_copy(cand_v, out_hbm.at[pl.ds(base + g * 8, 8)])


@functools.cache
def _gather_candidates():
    return pl.kernel(
        _gather_body,
        out_type=jax.ShapeDtypeStruct((NQ, CAND_W), jnp.float32),
        mesh=plsc.VectorSubcoreMesh(
            core_axis_name="c", subcore_axis_name="s",
            num_cores=_SC_NC, num_subcores=_SC_NS,
        ),
        scratch_types=[
            pltpu.VMEM((_RPW, IDS_PAD), jnp.int32),
            pltpu.VMEM((K, 8, _GW), jnp.float32),
            pltpu.VMEM((8, CAND_W), jnp.float32),
            pltpu.SemaphoreType.DMA,
        ],
    )


NCAND = K * CHUNK


def _final_body(cand_ref, ids_ref, out_ref):
    ids = ids_ref[...][:, :K]
    p_iota = lax.broadcasted_iota(jnp.int32, (QBLK, CAND_W), 1)
    v = jnp.where(p_iota < NCAND, cand_ref[...], _NEG)
    j_iota = lax.broadcasted_iota(jnp.int32, (QBLK, K), 1)
    outs = []
    for _ in range(K):
        m = jnp.max(v, axis=1)
        candp = jnp.where(v == m[:, None], p_iota, _BIG)
        p = jnp.min(candp, axis=1)
        jj = p // CHUNK
        lane = p - jj * CHUNK
        cidsel = jnp.sum(jnp.where(j_iota == jj[:, None], ids, 0), axis=1)
        outs.append(cidsel * CHUNK + lane)
        v = jnp.where(p_iota == p[:, None], _NEG, v)
    out_ref[...] = jnp.stack(outs, axis=1)


def _final_topk(cand, ids):
    return pl.pallas_call(
        _final_body,
        grid=(NQBLK,),
        in_specs=[
            pl.BlockSpec((QBLK, CAND_W), lambda qi: (qi, 0)),
            pl.BlockSpec((QBLK, IDS_PAD), lambda qi: (qi, 0)),
        ],
        out_specs=pl.BlockSpec((QBLK, K), lambda qi: (qi, 0)),
        out_shape=jax.ShapeDtypeStruct((NQ, K), jnp.int32),
    )(cand, ids)


def kernel(query_embeddings, W, b, doc_embeddings):
    qn = _project(query_embeddings, W, b.reshape(1, D))
    sim, mx = _sim_and_maxima(qn, doc_embeddings)
    ids = _select_chunks(mx)
    cand = _gather_candidates()(sim, ids)
    topk = _final_topk(cand, ids)
    return (topk, sim)


# speedup vs baseline: 1.4293x; 1.0151x over previous
import functools

import jax
import jax.numpy as jnp
from jax import lax
from jax.experimental import pallas as pl
from jax.experimental.pallas import tpu as pltpu
from jax.experimental.pallas import tpu_sc as plsc

NQ = 1024
SEQ = 20
D = 128
ND = 100000
K = 10

CHUNK = 160
NCHUNK = ND // CHUNK
QBLK = 1024
NQBLK = NQ // QBLK
DBLK = 5120
CPB = DBLK // CHUNK
NDBLK = (ND + DBLK - 1) // DBLK
NCHUNK_PAD = NDBLK * CPB

_BIG = 1 << 30
_NEG = -jnp.inf
IDS_PAD = 128


def _project_body(q_ref, w_ref, b_ref, qn_ref):
    pooled = jnp.mean(q_ref[...], axis=1)
    proj = lax.dot_general(pooled, w_ref[...], (((1,), (1,)), ((), ())))
    proj = proj + b_ref[...]
    nrm = jnp.sqrt(jnp.sum(proj * proj, axis=1, keepdims=True))
    nrm = jnp.maximum(nrm, 1e-12)
    qn_ref[...] = proj / nrm


def _project(q, w, b2d):
    return pl.pallas_call(
        _project_body,
        grid=(NQBLK,),
        in_specs=[
            pl.BlockSpec((QBLK, SEQ, D), lambda i: (i, 0, 0)),
            pl.BlockSpec((D, D), lambda i: (0, 0)),
            pl.BlockSpec((1, D), lambda i: (0, 0)),
        ],
        out_specs=pl.BlockSpec((QBLK, D), lambda i: (i, 0)),
        out_shape=jax.ShapeDtypeStruct((NQ, D), jnp.float32),
    )(q, w, b2d)


def _sim_body(qn_ref, doc_ref, sim_ref, mx_ref):
    di = pl.program_id(0)
    s = lax.dot_general(qn_ref[...], doc_ref[...], (((1,), (1,)), ((), ())))
    sim_ref[...] = s
    lane = lax.broadcasted_iota(jnp.int32, (QBLK, DBLK), 1)
    valid = (di * DBLK + lane) < ND
    sm = jnp.where(valid, s, _NEG)
    maxes = [jnp.max(sm[:, c * CHUNK:(c + 1) * CHUNK], axis=1)
             for c in range(CPB)]
    mx_ref[...] = jnp.stack(maxes, axis=0)


def _sim_and_maxima(qn, doc):
    return pl.pallas_call(
        _sim_body,
        grid=(NDBLK, NQBLK),
        in_specs=[
            pl.BlockSpec((QBLK, D), lambda di, qi: (qi, 0)),
            pl.BlockSpec((DBLK, D), lambda di, qi: (di, 0)),
        ],
        out_specs=[
            pl.BlockSpec((QBLK, DBLK), lambda di, qi: (qi, di)),
            pl.BlockSpec((CPB, QBLK), lambda di, qi: (di, qi)),
        ],
        out_shape=[
            jax.ShapeDtypeStruct((NQ, ND), jnp.float32),
            jax.ShapeDtypeStruct((NCHUNK_PAD, NQ), jnp.float32),
        ],
    )(qn, doc)


def _select_body(mx_ref, ids_ref):
    v = mx_ref[...]
    cid = lax.broadcasted_iota(jnp.int32, v.shape, 0)
    picks = []
    for _ in range(K):
        m = jnp.max(v, axis=0)
        cand = jnp.where(v == m[None, :], cid, _BIG)
        sel = jnp.min(cand, axis=0)
        picks.append(sel)
        v = jnp.where(cid == sel[None, :], _NEG, v)
    idmat = jnp.stack(picks, axis=0)
    outs = []
    for _ in range(K):
        mn = jnp.min(idmat, axis=0)
        outs.append(mn)
        idmat = jnp.where(idmat == mn[None, :], _BIG, idmat)
    sorted_ids = jnp.stack(outs, axis=1)
    pad = jnp.zeros((QBLK, IDS_PAD - K), jnp.int32)
    ids_ref[...] = jnp.concatenate([sorted_ids, pad], axis=1)


def _select_chunks(mx):
    return pl.pallas_call(
        _select_body,
        grid=(NQBLK,),
        in_specs=[pl.BlockSpec((NCHUNK_PAD, QBLK), lambda qi: (0, qi))],
        out_specs=pl.BlockSpec((QBLK, IDS_PAD), lambda qi: (qi, 0)),
        out_shape=jax.ShapeDtypeStruct((NQ, IDS_PAD), jnp.int32),
    )(mx)


_SC_NC = 2
_SC_NS = 16
_NW = _SC_NC * _SC_NS
_B = NQ * K
_BPW = _B // _NW
_GCHUNKS = [(o, min(128, _BPW - o)) for o in range(0, _BPW, 128)]


_RPW = NQ // _NW


_GW = 256


NCAND = K * CHUNK
CAND_W = 1792


def _gather_body(sim_hbm, ids_hbm, out_hbm, ids_v, buf8_v, cand_v, sem):
    c = lax.axis_index("c")
    s = lax.axis_index("s")
    wid = s * _SC_NC + c
    base = wid * _RPW
    pltpu.sync_copy(ids_hbm.at[pl.ds(base, _RPW)], ids_v)

    for g in range(_RPW // 8):
        def row(r8, carry):
            r_loc = g * 8 + r8
            row0 = pl.multiple_of(base + g * 8, 8)
            rvec = ids_v[r_loc, pl.ds(0, 16)]
            rems = []
            cps = []
            for j in range(K):
                q = rvec[j] * CHUNK
                off_al = pl.multiple_of((q // 128) * 128, 128)
                rems.append(q - off_al)
                cps.append(pltpu.make_async_copy(
                    sim_hbm.at[pl.ds(row0, 8), pl.ds(off_al, _GW)],
                    buf8_v.at[j],
                    sem,
                ))
            for cp in cps:
                cp.start()
            for cp in cps:
                cp.wait()
            for j in range(K):
                for v in range(CHUNK // 16):
                    cand_v[r8, pl.ds(j * CHUNK + v * 16, 16)] = (
                        buf8_v[j, r8, pl.ds(rems[j] + v * 16, 16)])
            return carry

        lax.fori_loop(0, 8, row, 0)
        pltpu.sync_copy(cand_v, out_hbm.at[pl.ds(base + g * 8, 8)])


@functools.cache
def _gather_candidates():
    return pl.kernel(
        _gather_body,
        out_type=jax.ShapeDtypeStruct((NQ, CAND_W), jnp.float32),
        mesh=plsc.VectorSubcoreMesh(
            core_axis_name="c", subcore_axis_name="s",
            num_cores=_SC_NC, num_subcores=_SC_NS,
        ),
        scratch_types=[
            pltpu.VMEM((_RPW, IDS_PAD), jnp.int32),
            pltpu.VMEM((K, 8, _GW), jnp.float32),
            pltpu.VMEM((8, CAND_W), jnp.float32),
            pltpu.SemaphoreType.DMA,
        ],
    )


NCAND = K * CHUNK


def _final_body(cand_ref, ids_ref, out_ref):
    ids = ids_ref[...][:, :K]
    p_iota = lax.broadcasted_iota(jnp.int32, (QBLK, CAND_W), 1)
    v = jnp.where(p_iota < NCAND, cand_ref[...], _NEG)
    j_iota = lax.broadcasted_iota(jnp.int32, (QBLK, K), 1)
    outs = []
    for _ in range(K):
        m = jnp.max(v, axis=1)
        candp = jnp.where(v == m[:, None], p_iota, _BIG)
        p = jnp.min(candp, axis=1)
        jj = p // CHUNK
        lane = p - jj * CHUNK
        cidsel = jnp.sum(jnp.where(j_iota == jj[:, None], ids, 0), axis=1)
        outs.append(cidsel * CHUNK + lane)
        v = jnp.where(p_iota == p[:, None], _NEG, v)
    out_ref[...] = jnp.stack(outs, axis=1)


def _final_topk(cand, ids):
    return pl.pallas_call(
        _final_body,
        grid=(NQBLK,),
        in_specs=[
            pl.BlockSpec((QBLK, CAND_W), lambda qi: (qi, 0)),
            pl.BlockSpec((QBLK, IDS_PAD), lambda qi: (qi, 0)),
        ],
        out_specs=pl.BlockSpec((QBLK, K), lambda qi: (qi, 0)),
        out_shape=jax.ShapeDtypeStruct((NQ, K), jnp.int32),
    )(cand, ids)


def kernel(query_embeddings, W, b, doc_embeddings):
    qn = _project(query_embeddings, W, b.reshape(1, D))
    sim, mx = _sim_and_maxima(qn, doc_embeddings)
    ids = _select_chunks(mx)
    cand = _gather_candidates()(sim, ids)
    topk = _final_topk(cand, ids)
    return (topk, sim)
